# Initial kernel scaffold; baseline (speedup 1.0000x reference)
import jax
import jax.numpy as jnp
from jax.experimental import pallas as pl


def kernel(x, edge_index, Ws, Wn, b, gamma, beta):
    raise NotImplementedError("write your pallas kernel here")


# speedup vs baseline: 6.8556x; 6.8556x over previous
import functools

import jax
import jax.numpy as jnp
from jax import lax
from jax.experimental import pallas as pl
from jax.experimental.pallas import tpu as pltpu
import jax.experimental.pallas.tpu_sc as plsc

N = 10000
E = 320000
D = 128
NE = 4
NC = 2
NS = 16
K = 80
CHUNKS_PER_TILE = E // NS // K
G = 50
NG = CHUNKS_PER_TILE // G
CHUNKS_PER_WORKER = CHUNKS_PER_TILE // NC
NPAD = 10240
ROWS_PER_TILE = NPAD // NS

_MESH = plsc.VectorSubcoreMesh(core_axis_name="c", subcore_axis_name="s")


@functools.partial(
    pl.kernel,
    out_type=(jax.ShapeDtypeStruct((NC, NPAD, D), jnp.float32),
              jax.ShapeDtypeStruct((NC, NPAD, D), jnp.float32)),
    mesh=_MESH,
    scratch_types=[
        pltpu.VMEM((CHUNKS_PER_WORKER, K), jnp.int32),
        pltpu.VMEM((CHUNKS_PER_WORKER, K), jnp.int32),
        pltpu.VMEM((K, D), jnp.float32),
        pltpu.VMEM_SHARED((NPAD, D), jnp.float32),
        pltpu.SemaphoreType.DMA,
    ],
)
def _sc_pass_a(x_hbm, srcb_hbm, dstb_hbm, ones_hbm, zeros_hbm,
               agg_hbm, deg_hbm, src_v, dst_v, rows_v, acc, sem):
    c = lax.axis_index("c")
    s = lax.axis_index("s")
    row0 = s * ROWS_PER_TILE
    pltpu.sync_copy(zeros_hbm.at[pl.ds(row0, ROWS_PER_TILE)],
                    acc.at[pl.ds(row0, ROWS_PER_TILE)])
    w = s * NC + c
    pltpu.sync_copy(srcb_hbm.at[w], src_v)
    pltpu.sync_copy(dstb_hbm.at[w], dst_v)
    plsc.subcore_barrier()

    def body(j, carry):
        pltpu.async_copy(x_hbm.at[src_v.at[j]], rows_v, sem).wait()
        pltpu.sync_copy(rows_v, acc.at[dst_v.at[j]], add=True)
        return carry

    lax.fori_loop(0, CHUNKS_PER_WORKER, body, 0)
    plsc.subcore_barrier()
    pltpu.sync_copy(acc.at[pl.ds(row0, ROWS_PER_TILE)],
                    agg_hbm.at[c, pl.ds(row0, ROWS_PER_TILE)])
    plsc.subcore_barrier()
    pltpu.sync_copy(zeros_hbm.at[pl.ds(row0, ROWS_PER_TILE)],
                    acc.at[pl.ds(row0, ROWS_PER_TILE)])
    pltpu.sync_copy(ones_hbm, rows_v)
    plsc.subcore_barrier()

    def body_deg(j, carry):
        pltpu.sync_copy(rows_v, acc.at[dst_v.at[j]], add=True)
        return carry

    lax.fori_loop(0, CHUNKS_PER_WORKER, body_deg, 0)
    plsc.subcore_barrier()
    pltpu.sync_copy(acc.at[pl.ds(row0, ROWS_PER_TILE)],
                    deg_hbm.at[c, pl.ds(row0, ROWS_PER_TILE)])


@functools.partial(
    pl.kernel,
    out_type=jax.ShapeDtypeStruct((NE * NPAD, D), jnp.float32),
    mesh=_MESH,
    scratch_types=[
        pltpu.VMEM((G, K), jnp.int32),
        pltpu.VMEM((G, K), jnp.int32),
        pltpu.VMEM((K, D), jnp.float32),
        pltpu.VMEM_SHARED((NPAD, D), jnp.float32),
        pltpu.SemaphoreType.DMA,
    ],
)
def _sc_pass_b(h1_hbm, srcb4_hbm, dstb_hbm, zeros_hbm, out_hbm,
               src_v, dst_v, rows_v, acc, sem):
    c = lax.axis_index("c")
    s = lax.axis_index("s")
    row0 = s * ROWS_PER_TILE
    for r in range(NE // NC):
        e = r * NC + c
        pltpu.sync_copy(zeros_hbm.at[pl.ds(row0, ROWS_PER_TILE)],
                        acc.at[pl.ds(row0, ROWS_PER_TILE)])
        plsc.subcore_barrier()

        def group(g, carry):
            pltpu.sync_copy(srcb4_hbm.at[(e * NS + s) * NG + g], src_v)
            pltpu.sync_copy(dstb_hbm.at[s * NG + g], dst_v)

            def body(j, carry2):
                pltpu.async_copy(h1_hbm.at[src_v.at[j]], rows_v, sem).wait()
                pltpu.sync_copy(rows_v, acc.at[dst_v.at[j]], add=True)
                return carry2

            lax.fori_loop(0, G, body, 0)
            return carry

        lax.fori_loop(0, NG, group, 0)
        plsc.subcore_barrier()
        pltpu.sync_copy(acc.at[pl.ds(row0, ROWS_PER_TILE)],
                        out_hbm.at[pl.ds(e * NPAD + row0, ROWS_PER_TILE)])
        plsc.subcore_barrier()


BM = 2000
NB = N // BM


def _tc_layer0_mm(x_ref, aggp_ref, degp_ref, ws_ref, wn_ref, b_ref,
                  z_ref, mom_ref):
    agg = aggp_ref[0] + aggp_ref[1]
    deg = jnp.maximum(degp_ref[0, :, 0:1] + degp_ref[1, :, 0:1], 1.0)
    magg = agg / deg
    xb = x_ref[...]
    moms = []
    for e in range(NE):
        z = (jnp.dot(xb, ws_ref[e], preferred_element_type=jnp.float32)
             + jnp.dot(magg, wn_ref[e], preferred_element_type=jnp.float32)
             + b_ref[e][None, :])
        z_ref[e] = z
        moms.append(jnp.sum(z, axis=0, keepdims=True))
        moms.append(jnp.sum(z * z, axis=0, keepdims=True))
    mom_ref[0] = jnp.concatenate(moms[0::2] + moms[1::2], axis=0)


def _tc_layer1_mm(h1_ref, agg1_ref, degp_ref, ws_ref, wn_ref, b_ref,
                  z_ref, mom_ref):
    deg = jnp.maximum(degp_ref[0, :, 0:1] + degp_ref[1, :, 0:1], 1.0)
    moms = []
    for e in range(NE):
        magg = agg1_ref[e] / deg
        z = (jnp.dot(h1_ref[e], ws_ref[e], preferred_element_type=jnp.float32)
             + jnp.dot(magg, wn_ref[e], preferred_element_type=jnp.float32)
             + b_ref[e][None, :])
        z_ref[e] = z
        moms.append(jnp.sum(z, axis=0, keepdims=True))
        moms.append(jnp.sum(z * z, axis=0, keepdims=True))
    mom_ref[0] = jnp.concatenate(moms[0::2] + moms[1::2], axis=0)


def _tc_bn_relu(z_ref, mom_ref, gb_ref, out_ref):
    m = jnp.sum(mom_ref[...], axis=0)
    for e in range(NE):
        mu = m[e] / N
        var = m[NE + e] / N - mu * mu
        inv = gb_ref[e] * lax.rsqrt(var + 1e-5)
        h = inv[None, :] * (z_ref[e] - mu[None, :]) + gb_ref[NE + e][None, :]
        out_ref[e] = jnp.maximum(h, 0.0)


def _full(shape):
    return pl.BlockSpec(shape, lambda i: (0,) * len(shape))


def _rows3(lead):
    return pl.BlockSpec((lead, BM, D), lambda i: (0, i, 0))


def _layer0_mm(x, aggp, degp, ws, wn, bias):
    return pl.pallas_call(
        _tc_layer0_mm,
        grid=(NB,),
        in_specs=[
            pl.BlockSpec((BM, D), lambda i: (i, 0)),
            pl.BlockSpec((NC, BM, D), lambda i: (0, i, 0)),
            pl.BlockSpec((NC, BM, D), lambda i: (0, i, 0)),
            _full((NE, D, D)),
            _full((NE, D, D)),
            _full((8, D)),
        ],
        out_specs=[_rows3(NE), pl.BlockSpec((1, 8, D), lambda i: (i, 0, 0))],
        out_shape=[
            jax.ShapeDtypeStruct((NE, N, D), jnp.float32),
            jax.ShapeDtypeStruct((NB, 8, D), jnp.float32),
        ],
    )(x, aggp, degp, ws, wn, bias)


def _layer1_mm(h1, agg1, degp, ws, wn, bias):
    return pl.pallas_call(
        _tc_layer1_mm,
        grid=(NB,),
        in_specs=[
            _rows3(NE),
            _rows3(NE),
            pl.BlockSpec((NC, BM, D), lambda i: (0, i, 0)),
            _full((NE, D, D)),
            _full((NE, D, D)),
            _full((8, D)),
        ],
        out_specs=[_rows3(NE), pl.BlockSpec((1, 8, D), lambda i: (i, 0, 0))],
        out_shape=[
            jax.ShapeDtypeStruct((NE, N, D), jnp.float32),
            jax.ShapeDtypeStruct((NB, 8, D), jnp.float32),
        ],
    )(h1, agg1, degp, ws, wn, bias)


def _bn_relu(z, mom, gb):
    return pl.pallas_call(
        _tc_bn_relu,
        grid=(NB,),
        in_specs=[_rows3(NE), _full((NB, 8, D)), _full((8, D))],
        out_specs=_rows3(NE),
        out_shape=jax.ShapeDtypeStruct((NE, N, D), jnp.float32),
    )(z, mom, gb)


def kernel(x, edge_index, Ws, Wn, b, gamma, beta):
    src = edge_index[0].astype(jnp.int32)
    dst = edge_index[1].astype(jnp.int32)
    srcb_a = src.reshape(NC * NS, CHUNKS_PER_WORKER, K)
    dstb_a = dst.reshape(NC * NS, CHUNKS_PER_WORKER, K)
    dstb = dst.reshape(NS * NG, G, K)
    offs = (jnp.arange(NE, dtype=jnp.int32) * N)[:, None]
    srcb4 = (src[None, :] + offs).reshape(NE * NS * NG, G, K)

    ones_k = jnp.ones((K, D), jnp.float32)
    zeros_d = jnp.zeros((NPAD, D), jnp.float32)

    pad4 = jnp.zeros((NE, D), jnp.float32)
    bias0 = jnp.concatenate([b[:, 0], pad4], axis=0)
    bias1 = jnp.concatenate([b[:, 1], pad4], axis=0)
    gb0 = jnp.concatenate([gamma[:, 0], beta[:, 0]], axis=0)
    gb1 = jnp.concatenate([gamma[:, 1], beta[:, 1]], axis=0)

    aggp, degp = _sc_pass_a(x, srcb_a, dstb_a, ones_k, zeros_d)
    z0, mom0 = _layer0_mm(x, aggp, degp, Ws[:, 0], Wn[:, 0], bias0)
    h1 = _bn_relu(z0, mom0, gb0)

    agg1 = _sc_pass_b(h1.reshape(NE * N, D), srcb4, dstb, zeros_d)
    z1, mom1 = _layer1_mm(h1, agg1.reshape(NE, NPAD, D), degp,
                          Ws[:, 1], Wn[:, 1], bias1)
    h2 = _bn_relu(z1, mom1, gb1)

    return jnp.transpose(h2, (1, 2, 0))
